# Initial kernel scaffold; baseline (speedup 1.0000x reference)
import jax
import jax.numpy as jnp
from jax.experimental import pallas as pl


def kernel(hidden_states, topk_weights, topk_idx, w_gate, w_up, w_down):
    raise NotImplementedError("write your pallas kernel here")


# speedup vs baseline: 1.3075x; 1.3075x over previous
import functools

import jax
import jax.numpy as jnp
from jax.experimental import pallas as pl
from jax.experimental.pallas import tpu as pltpu

T = 2048
D = 1024
F = 1024
E = 8
K = 2
TB = 256


def _moe_dense_body(idx_ref, tw_ref, x_ref, wg_ref, wu_ref, wd_ref, o_ref):
    e = pl.program_id(1)
    x = x_ref[...]
    g = jnp.dot(x, wg_ref[0], preferred_element_type=jnp.float32)
    u = jnp.dot(x, wu_ref[0], preferred_element_type=jnp.float32)
    h = g * jax.nn.sigmoid(g) * u
    y = jnp.dot(h, wd_ref[0], preferred_element_type=jnp.float32)

    idx = idx_ref[...]
    tw = tw_ref[...]
    tw = tw / jnp.sum(tw, axis=1, keepdims=True)
    c = jnp.zeros((TB, 1), jnp.float32)
    for k in range(K):
        c = c + jnp.where(idx[:, k:k + 1] == e, tw[:, k:k + 1], 0.0)
    contrib = y * c

    @pl.when(e == 0)
    def _init():
        o_ref[...] = contrib

    @pl.when(e != 0)
    def _acc():
        o_ref[...] += contrib


def kernel(hidden_states, topk_weights, topk_idx, w_gate, w_up, w_down):
    grid = (T // TB, E)
    return pl.pallas_call(
        _moe_dense_body,
        grid=grid,
        in_specs=[
            pl.BlockSpec((TB, K), lambda t, e: (t, 0)),
            pl.BlockSpec((TB, K), lambda t, e: (t, 0)),
            pl.BlockSpec((TB, D), lambda t, e: (t, 0)),
            pl.BlockSpec((1, D, F), lambda t, e: (e, 0, 0)),
            pl.BlockSpec((1, D, F), lambda t, e: (e, 0, 0)),
            pl.BlockSpec((1, F, D), lambda t, e: (e, 0, 0)),
        ],
        out_specs=pl.BlockSpec((TB, D), lambda t, e: (t, 0)),
        out_shape=jax.ShapeDtypeStruct((T, D), jnp.float32),
        compiler_params=pltpu.CompilerParams(
            dimension_semantics=("parallel", "arbitrary"),
        ),
    )(topk_idx, topk_weights, hidden_states, w_gate, w_up, w_down)
